# baseline (speedup 1.0000x reference)
import functools

import jax
import jax.numpy as jnp
from jax import lax
from jax.experimental import pallas as pl
from jax.experimental.pallas import tpu as pltpu
from jax.experimental.pallas import tpu_sc as plsc

NUM_TABLES = 26
VOCAB = 100000
DIM = 32
BATCH = 16384

NC = 2
NS = 16
L = 16
NW = NC * NS

CHUNK = 1024
GATHER = 128
CHUNKS_PER_TABLE = BATCH // CHUNK
TOTAL_CHUNKS = NUM_TABLES * CHUNKS_PER_TABLE
CHUNKS_PER_W = TOTAL_CHUNKS // NW
GATHERS_PER_CHUNK = CHUNK // GATHER


def _sc_lookup(ids_flat, tables_flat):
    mesh = plsc.VectorSubcoreMesh(core_axis_name="c", subcore_axis_name="s")

    @functools.partial(
        pl.kernel,
        mesh=mesh,
        compiler_params=pltpu.CompilerParams(use_tc_tiling_on_sc=False),
        out_type=jax.ShapeDtypeStruct((NUM_TABLES * BATCH, DIM), jnp.float32),
        scratch_types=[
            pltpu.VMEM((GATHERS_PER_CHUNK, GATHER), jnp.int32),
            pltpu.VMEM((CHUNK, DIM), jnp.float32),
            pltpu.SemaphoreType.DMA,
        ],
    )
    def k(ids_hbm, tab_hbm, out_hbm, idx_v, rows_v, sem):
        wid = lax.axis_index("s") * NC + lax.axis_index("c")

        def body(j, carry):
            c = wid * CHUNKS_PER_W + j
            base = c * CHUNK
            t = c // CHUNKS_PER_TABLE
            off = t * VOCAB
            id_copies = []
            for g in range(GATHERS_PER_CHUNK):
                id_copies.append(
                    pltpu.make_async_copy(
                        ids_hbm.at[pl.ds(base + g * GATHER, GATHER)],
                        idx_v.at[g],
                        sem,
                    )
                )
                id_copies[-1].start()
            for cp in id_copies:
                cp.wait()

            def addoff(i, carry2):
                r = i // (GATHER // L)
                col = (i % (GATHER // L)) * L
                idx_v[r, pl.ds(col, L)] = idx_v[r, pl.ds(col, L)] + off
                return carry2

            lax.fori_loop(0, CHUNK // L, addoff, 0)

            copies = []
            for g in range(GATHERS_PER_CHUNK):
                copies.append(
                    pltpu.make_async_copy(
                        tab_hbm.at[idx_v.at[g]],
                        rows_v.at[pl.ds(g * GATHER, GATHER)],
                        sem,
                    )
                )
                copies[-1].start()
            for cp in copies:
                cp.wait()
            pltpu.sync_copy(rows_v, out_hbm.at[pl.ds(base, CHUNK)])
            return carry

        lax.fori_loop(0, CHUNKS_PER_W, body, 0)

    return k(ids_flat, tables_flat)


def kernel(ids, tables):
    out_flat = _sc_lookup(
        ids.reshape(NUM_TABLES * BATCH),
        tables.reshape(NUM_TABLES * VOCAB, DIM),
    )
    return out_flat.reshape(NUM_TABLES, BATCH, DIM)
